# baseline (speedup 1.0000x reference)
import functools

import jax
import jax.numpy as jnp
from jax import lax
from jax.experimental import pallas as pl
from jax.experimental.pallas import tpu as pltpu
from jax.experimental.pallas import tpu_sc as plsc

N_NODES = 10000
N_EDGES = 320000
ND = 128
NLAYERS = 3
EPS_BN = 1e-5

NC = 2
NS = 16
HD = ND // NC
CHUNK = 128
CHUNKS_PER_TILE = 158
E_PAD = NS * CHUNKS_PER_TILE * CHUNK
ACC_ROWS = 10112
ROWS_PER_TILE = ACC_ROWS // NS


def _agg_body(h_hbm, src_hbm, dst_hbm, out_hbm,
              acc_sh, src_v, dst_v, buf0, buf1, zbuf, gsem0a, gsem0b, gsem1a, gsem1b):
    H = CHUNK // 2
    c = lax.axis_index("c")
    s = lax.axis_index("s")

    pltpu.sync_copy(src_hbm.at[c, s], src_v)
    pltpu.sync_copy(dst_hbm.at[s], dst_v)

    def issue(j, buf, sa, sb):
        pltpu.async_copy(h_hbm.at[src_v.at[j, pl.ds(0, H)]], buf.at[pl.ds(0, H)], sa)
        pltpu.async_copy(h_hbm.at[src_v.at[j, pl.ds(H, H)]], buf.at[pl.ds(H, H)], sb)

    def drain(j, buf, sa, sb):
        pltpu.make_async_copy(h_hbm.at[src_v.at[j, pl.ds(0, H)]],
                              buf.at[pl.ds(0, H)], sa).wait()
        pltpu.make_async_copy(h_hbm.at[src_v.at[j, pl.ds(H, H)]],
                              buf.at[pl.ds(H, H)], sb).wait()

    issue(0, buf0, gsem0a, gsem0b)
    issue(1, buf1, gsem1a, gsem1b)

    def zrow(r, carry):
        for jj in range(HD // 16):
            zbuf[r, pl.ds(jj * 16, 16)] = jnp.zeros((16,), jnp.float32)
        return carry
    lax.fori_loop(0, 16, zrow, 0)
    base = s * ROWS_PER_TILE

    def zslice(k, carry):
        pltpu.sync_copy(zbuf, acc_sh.at[pl.ds(base + k * 16, 16)])
        return carry
    lax.fori_loop(0, ROWS_PER_TILE // 16, zslice, 0)
    rem = ROWS_PER_TILE % 16
    if rem:
        pltpu.sync_copy(zbuf.at[pl.ds(0, rem)],
                        acc_sh.at[pl.ds(base + ROWS_PER_TILE - rem, rem)])
    plsc.subcore_barrier()

    def body(i, carry):
        j0 = 2 * i
        drain(j0, buf0, gsem0a, gsem0b)
        pltpu.sync_copy(buf0, acc_sh.at[dst_v.at[j0]], add=True)
        issue(j0 + 2, buf0, gsem0a, gsem0b)
        j1 = j0 + 1
        drain(j1, buf1, gsem1a, gsem1b)
        pltpu.sync_copy(buf1, acc_sh.at[dst_v.at[j1]], add=True)
        issue(j1 + 2, buf1, gsem1a, gsem1b)
        return carry
    lax.fori_loop(0, CHUNKS_PER_TILE // 2, body, 0)

    drain(CHUNKS_PER_TILE, buf0, gsem0a, gsem0b)
    drain(CHUNKS_PER_TILE + 1, buf1, gsem1a, gsem1b)
    plsc.subcore_barrier()

    pltpu.sync_copy(acc_sh.at[pl.ds(base, ROWS_PER_TILE)],
                    out_hbm.at[c, pl.ds(base, ROWS_PER_TILE)])


@functools.cache
def _agg_kernel():
    return pl.kernel(
        _agg_body,
        out_type=jax.ShapeDtypeStruct((NC, ACC_ROWS, HD), jnp.float32),
        mesh=plsc.VectorSubcoreMesh(core_axis_name="c", subcore_axis_name="s",
                                    num_cores=NC, num_subcores=NS),
        compiler_params=pltpu.CompilerParams(use_tc_tiling_on_sc=False),
        scratch_types=(
            [pltpu.VMEM_SHARED((ACC_ROWS, HD), jnp.float32)]
            + [pltpu.VMEM((CHUNKS_PER_TILE + 2, CHUNK), jnp.int32),
               pltpu.VMEM((CHUNKS_PER_TILE, CHUNK), jnp.int32)]
            + [pltpu.VMEM((CHUNK, HD), jnp.float32)] * 2
            + [pltpu.VMEM((16, HD), jnp.float32)]
            + [pltpu.SemaphoreType.DMA] * 4
        ),
    )


def _dense_body(h_ref, agg_ref, w_ref, b_ref, g_ref, be_ref, o_ref):
    agg = jnp.concatenate(
        [agg_ref[0, :N_NODES, :], agg_ref[1, :N_NODES, :]], axis=1)
    hsum = h_ref[...] + agg
    z = jnp.dot(hsum, w_ref[...], preferred_element_type=jnp.float32) + b_ref[...]
    mean = jnp.mean(z, axis=0, keepdims=True)
    zc = z - mean
    var = jnp.mean(zc * zc, axis=0, keepdims=True)
    out = zc * lax.rsqrt(var + EPS_BN) * g_ref[...] + be_ref[...]
    o_ref[...] = jnp.maximum(out, 0.0)


_dense_kernel = pl.pallas_call(
    _dense_body,
    out_shape=jax.ShapeDtypeStruct((N_NODES, ND), jnp.float32),
)


def kernel(x, edge_index, Ws, bs, gammas, betas):
    src = edge_index[0].astype(jnp.int32)
    dst = edge_index[1].astype(jnp.int32)
    pad = E_PAD - N_EDGES
    src_p = jnp.concatenate([src, jnp.zeros((pad,), jnp.int32)])
    dst_p = jnp.concatenate([dst, jnp.full((pad,), N_NODES, jnp.int32)])
    src_slabs = jnp.stack(
        [(2 * src_p).reshape(NS, CHUNKS_PER_TILE, CHUNK),
         (2 * src_p + 1).reshape(NS, CHUNKS_PER_TILE, CHUNK)])
    src_slabs = jnp.concatenate(
        [src_slabs, jnp.zeros((NC, NS, 2, CHUNK), jnp.int32)], axis=2)
    dst_slabs = dst_p.reshape(NS, CHUNKS_PER_TILE, CHUNK)

    h = x
    for l in range(NLAYERS):
        h2 = h.reshape(NC * N_NODES, HD)
        agg = _agg_kernel()(h2, src_slabs, dst_slabs)
        h = _dense_kernel(h, agg, Ws[l], bs[l].reshape(1, ND),
                          gammas[l].reshape(1, ND), betas[l].reshape(1, ND))
    return h


# speedup vs baseline: 1.0049x; 1.0049x over previous
import functools

import jax
import jax.numpy as jnp
from jax import lax
from jax.experimental import pallas as pl
from jax.experimental.pallas import tpu as pltpu
from jax.experimental.pallas import tpu_sc as plsc

N_NODES = 10000
N_EDGES = 320000
ND = 128
NLAYERS = 3
EPS_BN = 1e-5

NC = 2
NS = 16
HD = ND // NC
CHUNK = 128
CHUNKS_PER_TILE = 158
E_PAD = NS * CHUNKS_PER_TILE * CHUNK
ACC_ROWS = 10112
ROWS_PER_TILE = ACC_ROWS // NS


def _agg_body(h_hbm, src_hbm, dst_hbm, out_hbm,
              acc_sh, src_v, dst_v, buf0, buf1,
              gsem0a, gsem0b, gsem0c, gsem0d, gsem1a, gsem1b, gsem1c, gsem1d):
    H = CHUNK // 4
    c = lax.axis_index("c")
    s = lax.axis_index("s")

    pltpu.sync_copy(src_hbm.at[c, s], src_v)
    pltpu.sync_copy(dst_hbm.at[s], dst_v)

    def zrow(r, carry):
        for jj in range(HD // 16):
            buf0[r, pl.ds(jj * 16, 16)] = jnp.zeros((16,), jnp.float32)
        return carry
    lax.fori_loop(0, CHUNK, zrow, 0)
    base = s * ROWS_PER_TILE
    for k in range(ROWS_PER_TILE // CHUNK):
        pltpu.sync_copy(buf0, acc_sh.at[pl.ds(base + k * CHUNK, CHUNK)])
    rem = ROWS_PER_TILE % CHUNK
    if rem:
        pltpu.sync_copy(buf0.at[pl.ds(0, rem)],
                        acc_sh.at[pl.ds(base + ROWS_PER_TILE - rem, rem)])
    plsc.subcore_barrier()

    def issue(j, buf, sems):
        for q in range(4):
            pltpu.async_copy(h_hbm.at[src_v.at[j, pl.ds(q * H, H)]],
                             buf.at[pl.ds(q * H, H)], sems[q])

    def drain(j, buf, sems):
        for q in range(4):
            pltpu.make_async_copy(h_hbm.at[src_v.at[j, pl.ds(q * H, H)]],
                                  buf.at[pl.ds(q * H, H)], sems[q]).wait()

    sems0 = (gsem0a, gsem0b, gsem0c, gsem0d)
    sems1 = (gsem1a, gsem1b, gsem1c, gsem1d)
    issue(0, buf0, sems0)
    issue(1, buf1, sems1)

    def body(i, carry):
        j0 = 2 * i
        drain(j0, buf0, sems0)
        pltpu.sync_copy(buf0, acc_sh.at[dst_v.at[j0]], add=True)
        issue(j0 + 2, buf0, sems0)
        j1 = j0 + 1
        drain(j1, buf1, sems1)
        pltpu.sync_copy(buf1, acc_sh.at[dst_v.at[j1]], add=True)
        issue(j1 + 2, buf1, sems1)
        return carry
    lax.fori_loop(0, CHUNKS_PER_TILE // 2, body, 0)

    drain(CHUNKS_PER_TILE, buf0, sems0)
    drain(CHUNKS_PER_TILE + 1, buf1, sems1)
    plsc.subcore_barrier()

    pltpu.sync_copy(acc_sh.at[pl.ds(base, ROWS_PER_TILE)],
                    out_hbm.at[c, pl.ds(base, ROWS_PER_TILE)])


@functools.cache
def _agg_kernel():
    return pl.kernel(
        _agg_body,
        out_type=jax.ShapeDtypeStruct((NC, ACC_ROWS, HD), jnp.float32),
        mesh=plsc.VectorSubcoreMesh(core_axis_name="c", subcore_axis_name="s",
                                    num_cores=NC, num_subcores=NS),
        compiler_params=pltpu.CompilerParams(use_tc_tiling_on_sc=False),
        scratch_types=(
            [pltpu.VMEM_SHARED((ACC_ROWS, HD), jnp.float32)]
            + [pltpu.VMEM((CHUNKS_PER_TILE + 2, CHUNK), jnp.int32),
               pltpu.VMEM((CHUNKS_PER_TILE, CHUNK), jnp.int32)]
            + [pltpu.VMEM((CHUNK, HD), jnp.float32)] * 2
            + [pltpu.SemaphoreType.DMA] * 8
        ),
    )


def _dense_body(h_ref, agg_ref, w_ref, b_ref, g_ref, be_ref, o_ref):
    agg = jnp.concatenate(
        [agg_ref[0, :N_NODES, :], agg_ref[1, :N_NODES, :]], axis=1)
    hsum = h_ref[...] + agg
    z = jnp.dot(hsum, w_ref[...], preferred_element_type=jnp.float32) + b_ref[...]
    mean = jnp.mean(z, axis=0, keepdims=True)
    zc = z - mean
    var = jnp.mean(zc * zc, axis=0, keepdims=True)
    out = zc * lax.rsqrt(var + EPS_BN) * g_ref[...] + be_ref[...]
    o_ref[...] = jnp.maximum(out, 0.0)


_dense_kernel = pl.pallas_call(
    _dense_body,
    out_shape=jax.ShapeDtypeStruct((N_NODES, ND), jnp.float32),
)


def kernel(x, edge_index, Ws, bs, gammas, betas):
    src = edge_index[0].astype(jnp.int32)
    dst = edge_index[1].astype(jnp.int32)
    pad = E_PAD - N_EDGES
    src_p = jnp.concatenate([src, jnp.zeros((pad,), jnp.int32)])
    dst_p = jnp.concatenate([dst, jnp.full((pad,), N_NODES, jnp.int32)])
    src_slabs = jnp.stack(
        [(2 * src_p).reshape(NS, CHUNKS_PER_TILE, CHUNK),
         (2 * src_p + 1).reshape(NS, CHUNKS_PER_TILE, CHUNK)])
    src_slabs = jnp.concatenate(
        [src_slabs, jnp.zeros((NC, NS, 2, CHUNK), jnp.int32)], axis=2)
    dst_slabs = dst_p.reshape(NS, CHUNKS_PER_TILE, CHUNK)

    h = x
    for l in range(NLAYERS):
        h2 = h.reshape(NC * N_NODES, HD)
        agg = _agg_kernel()(h2, src_slabs, dst_slabs)
        h = _dense_kernel(h, agg, Ws[l], bs[l].reshape(1, ND),
                          gammas[l].reshape(1, ND), betas[l].reshape(1, ND))
    return h
